# baseline (speedup 1.0000x reference)
import jax
import jax.numpy as jnp
from jax import lax
from jax.experimental import pallas as pl
from jax.experimental.pallas import tpu as pltpu
from jax.experimental.pallas import tpu_sc as plsc

N = 10000
E = 320000
D = 128
H = 128
C = 64

NC = 2
NS = 16
NPAD = 10240
CHUNK = 40
NBUF = 4
EPAD = 327680
EPC = EPAD // NC
EPT = EPC // NS
NCHUNK = EPT // CHUNK
CPC = EPC // CHUNK
RPT = NPAD // NS
DEGW = 8

f32 = jnp.float32


def _make_agg(with_deg: bool):
  mesh = plsc.VectorSubcoreMesh(
      core_axis_name="c", subcore_axis_name="s",
      num_cores=NC, num_subcores=NS)

  out_type = [jax.ShapeDtypeStruct((NC * NPAD, D), f32)]
  if with_deg:
    out_type.append(jax.ShapeDtypeStruct((NC * NPAD, DEGW), f32))

  scratch = [
      pltpu.VMEM((NCHUNK, CHUNK), jnp.int32),
      pltpu.VMEM((NCHUNK, CHUNK), jnp.int32),
  ]
  scratch += [pltpu.VMEM((CHUNK, D), f32) for _ in range(NBUF)]
  scratch += [pltpu.VMEM_SHARED((NPAD, D), f32)]
  scratch += [pltpu.SemaphoreType.DMA for _ in range(NBUF)]
  scratch += [pltpu.SemaphoreType.DMA for _ in range(NBUF)]
  if with_deg:
    scratch += [
        pltpu.VMEM((CHUNK, DEGW), f32),
        pltpu.VMEM_SHARED((NPAD, DEGW), f32),
    ]
    scratch += [pltpu.SemaphoreType.DMA for _ in range(NBUF)]

  def body(*refs):
    if with_deg:
      feat, src2d, dst2d, zacc, zdacc, odeg, out_acc, out_deg = refs[:8]
      rest = refs[8:]
    else:
      feat, src2d, dst2d, zacc, out_acc = refs[:5]
      rest = refs[5:]
    sidx, didx = rest[0], rest[1]
    rows = list(rest[2:2 + NBUF])
    acc = rest[2 + NBUF]
    sg = list(rest[3 + NBUF:3 + 2 * NBUF])
    ss = list(rest[3 + 2 * NBUF:3 + 3 * NBUF])
    if with_deg:
      ones = rest[3 + 3 * NBUF]
      dacc = rest[4 + 3 * NBUF]
      sd = list(rest[5 + 3 * NBUF:5 + 4 * NBUF])

    c = lax.axis_index("c")
    s = lax.axis_index("s")
    rbase = s * RPT

    cb = c * CPC + s * NCHUNK
    init = [
        pltpu.async_copy(src2d.at[pl.ds(cb, NCHUNK)], sidx, sg[0]),
        pltpu.async_copy(dst2d.at[pl.ds(cb, NCHUNK)], didx, sg[1]),
        pltpu.async_copy(zacc, acc.at[pl.ds(rbase, RPT)], ss[0]),
    ]
    if with_deg:
      init.append(pltpu.async_copy(zdacc, dacc.at[pl.ds(rbase, RPT)], ss[1]))
      init.append(pltpu.async_copy(odeg, ones, sd[0]))
    for cp in init:
      cp.wait()
    plsc.subcore_barrier()

    def gather(j, b):
      return pltpu.async_copy(feat.at[sidx.at[j]], rows[b], sg[b])

    def gather_wait(j, b):
      pltpu.make_async_copy(feat.at[sidx.at[j]], rows[b], sg[b]).wait()

    def scatter(j, b):
      pltpu.async_copy(rows[b], acc.at[didx.at[j]], ss[b], add=True)
      if with_deg:
        pltpu.async_copy(ones, dacc.at[didx.at[j]], sd[b], add=True)

    def scatter_wait(j, b):
      pltpu.make_async_copy(rows[b], acc.at[didx.at[j]], ss[b]).wait()
      if with_deg:
        pltpu.make_async_copy(ones, dacc.at[didx.at[j]], sd[b]).wait()

    for b in range(NBUF):
      gather(b, b)

    def group(g, carry):
      j0 = g * NBUF
      for b in range(NBUF):
        gather_wait(j0 + b, b)
        scatter(j0 + b, b)
      for b in range(NBUF):
        scatter_wait(j0 + b, b)
        gather(j0 + NBUF + b, b)
      return carry

    lax.fori_loop(0, NCHUNK // NBUF - 1, group, 0)

    j0 = NCHUNK - NBUF
    for b in range(NBUF):
      gather_wait(j0 + b, b)
      scatter(j0 + b, b)
    for b in range(NBUF):
      scatter_wait(j0 + b, b)
    plsc.subcore_barrier()

    fin = [pltpu.async_copy(acc.at[pl.ds(rbase, RPT)],
                            out_acc.at[pl.ds(c * NPAD + rbase, RPT)], sg[0])]
    if with_deg:
      fin.append(pltpu.async_copy(dacc.at[pl.ds(rbase, RPT)],
                                  out_deg.at[pl.ds(c * NPAD + rbase, RPT)],
                                  sg[1]))
    for cp in fin:
      cp.wait()

  return pl.kernel(
      body, out_type=out_type, mesh=mesh, scratch_types=scratch,
      compiler_params=pltpu.CompilerParams(use_tc_tiling_on_sc=False))


_agg_deg = _make_agg(with_deg=True)
_agg = _make_agg(with_deg=False)


ROWS = 10000


def _l1_body(x, a0, a1, d0, d1, ws, wn, b, o):
  deg = d0[...][:, 0:1] + d1[...][:, 0:1]
  inv = 1.0 / jnp.maximum(deg, 1.0)
  hn = (a0[...] + a1[...]) * inv
  s0 = jnp.dot(x[...], ws[...], preferred_element_type=f32)
  o[...] = jnp.maximum(
      s0 + jnp.dot(hn, wn[...], preferred_element_type=f32) + b[...], 0.0)


def _l2_body(h1, a0, a1, d0, d1, ws, wn, b, wfc, bfc, o):
  deg = d0[...][:, 0:1] + d1[...][:, 0:1]
  inv = 1.0 / jnp.maximum(deg, 1.0)
  hn = (a0[...] + a1[...]) * inv
  h2 = jnp.maximum(
      jnp.dot(h1[...], ws[...], preferred_element_type=f32)
      + jnp.dot(hn, wn[...], preferred_element_type=f32) + b[...], 0.0)
  o[...] = jnp.dot(h2, wfc[...], preferred_element_type=f32) + bfc[...]


def _row_spec(w):
  return pl.BlockSpec((ROWS, w), lambda i: (i, 0))


def _full_spec(shape):
  return pl.BlockSpec(shape, lambda i: tuple(0 for _ in shape))


def _tc_layer1(x, a0, a1, d0, d1, ws, wn, b):
  return pl.pallas_call(
      _l1_body,
      grid=(N // ROWS,),
      in_specs=[
          _row_spec(D), _row_spec(D), _row_spec(D),
          _row_spec(DEGW), _row_spec(DEGW),
          _full_spec((D, H)), _full_spec((D, H)), _full_spec((1, H)),
      ],
      out_specs=_row_spec(H),
      out_shape=jax.ShapeDtypeStruct((N, H), f32),
  )(x, a0, a1, d0, d1, ws, wn, b)


def _tc_layer2(h1, a0, a1, d0, d1, ws, wn, b, wfc, bfc):
  return pl.pallas_call(
      _l2_body,
      grid=(N // ROWS,),
      in_specs=[
          _row_spec(H), _row_spec(H), _row_spec(H),
          _row_spec(DEGW), _row_spec(DEGW),
          _full_spec((H, H)), _full_spec((H, H)), _full_spec((1, H)),
          _full_spec((H, C)), _full_spec((1, C)),
      ],
      out_specs=_row_spec(C),
      out_shape=jax.ShapeDtypeStruct((N, C), f32),
  )(h1, a0, a1, d0, d1, ws, wn, b, wfc, bfc)


def kernel(x, edge_index, W_self0, W_neigh0, b0, W_self1, W_neigh1, b1,
           W_fc, b_fc):
  src = edge_index[0]
  dst = edge_index[1]
  pad = EPAD - E
  iota = jnp.arange(pad, dtype=dst.dtype)
  src2d = jnp.concatenate([src, iota % N]).reshape(-1, CHUNK)
  dst2d = jnp.concatenate([dst, N + iota % (NPAD - N)]).reshape(-1, CHUNK)
  zacc = jnp.zeros((RPT, D), f32)
  zdacc = jnp.zeros((RPT, DEGW), f32)
  odeg = jnp.ones((CHUNK, DEGW), f32)

  aggx, deg = _agg_deg(x, src2d, dst2d, zacc, zdacc, odeg)
  h1 = _tc_layer1(x, aggx[:N], aggx[NPAD:NPAD + N], deg[:N],
                  deg[NPAD:NPAD + N], W_self0, W_neigh0, b0.reshape(1, H))
  aggh, = _agg(h1, src2d, dst2d, zacc)
  out = _tc_layer2(h1, aggh[:N], aggh[NPAD:NPAD + N], deg[:N],
                   deg[NPAD:NPAD + N], W_self1, W_neigh1, b1.reshape(1, H),
                   W_fc, b_fc.reshape(1, C))
  return out


# speedup vs baseline: 1.0081x; 1.0081x over previous
import jax
import jax.numpy as jnp
from jax import lax
from jax.experimental import pallas as pl
from jax.experimental.pallas import tpu as pltpu
from jax.experimental.pallas import tpu_sc as plsc

N = 10000
E = 320000
D = 128
H = 128
C = 64

NC = 2
NS = 16
NPAD = 10240
CHUNK = 40
NBUF = 4
EPAD = 327680
EPC = EPAD // NC
EPT = EPC // NS
NCHUNK = EPT // CHUNK
CPC = EPC // CHUNK
RPT = NPAD // NS
DEGW = 8

f32 = jnp.float32


def _make_agg(with_deg: bool):
  mesh = plsc.VectorSubcoreMesh(
      core_axis_name="c", subcore_axis_name="s",
      num_cores=NC, num_subcores=NS)

  out_type = [jax.ShapeDtypeStruct((NC * NPAD, D), f32)]
  if with_deg:
    out_type.append(jax.ShapeDtypeStruct((NC * NPAD, DEGW), f32))

  scratch = [
      pltpu.VMEM((NCHUNK, CHUNK), jnp.int32),
      pltpu.VMEM((NCHUNK, CHUNK), jnp.int32),
  ]
  scratch += [pltpu.VMEM((CHUNK, D), f32) for _ in range(NBUF)]
  scratch += [pltpu.VMEM_SHARED((NPAD, D), f32)]
  scratch += [pltpu.SemaphoreType.DMA for _ in range(NBUF)]
  scratch += [pltpu.SemaphoreType.DMA for _ in range(NBUF)]
  if with_deg:
    scratch += [
        pltpu.VMEM((CHUNK, DEGW), f32),
        pltpu.VMEM_SHARED((NPAD, DEGW), f32),
    ]
    scratch += [pltpu.SemaphoreType.DMA for _ in range(NBUF)]

  def body(*refs):
    if with_deg:
      feat, src2d, dst2d, zacc, zdacc, odeg, out_acc, out_deg = refs[:8]
      rest = refs[8:]
    else:
      feat, src2d, dst2d, zacc, out_acc = refs[:5]
      rest = refs[5:]
    sidx, didx = rest[0], rest[1]
    rows = list(rest[2:2 + NBUF])
    acc = rest[2 + NBUF]
    sg = list(rest[3 + NBUF:3 + 2 * NBUF])
    ss = list(rest[3 + 2 * NBUF:3 + 3 * NBUF])
    if with_deg:
      ones = rest[3 + 3 * NBUF]
      dacc = rest[4 + 3 * NBUF]
      sd = list(rest[5 + 3 * NBUF:5 + 4 * NBUF])

    c = lax.axis_index("c")
    s = lax.axis_index("s")
    rbase = s * RPT

    cb = c * CPC + s * NCHUNK
    init = [
        pltpu.async_copy(src2d.at[pl.ds(cb, NCHUNK)], sidx, sg[0]),
        pltpu.async_copy(dst2d.at[pl.ds(cb, NCHUNK)], didx, sg[1]),
        pltpu.async_copy(zacc, acc.at[pl.ds(rbase, RPT)], ss[0]),
    ]
    if with_deg:
      init.append(pltpu.async_copy(zdacc, dacc.at[pl.ds(rbase, RPT)], ss[1]))
      init.append(pltpu.async_copy(odeg, ones, sd[0]))
    for cp in init:
      cp.wait()
    plsc.subcore_barrier()

    def gather(j, b):
      return pltpu.async_copy(feat.at[sidx.at[j]], rows[b], sg[b])

    def gather_wait(j, b):
      pltpu.make_async_copy(feat.at[sidx.at[j]], rows[b], sg[b]).wait()

    def scatter(j, b):
      pltpu.async_copy(rows[b], acc.at[didx.at[j]], ss[b], add=True)
      if with_deg:
        pltpu.async_copy(ones, dacc.at[didx.at[j]], sd[b], add=True)

    def scatter_wait(j, b):
      pltpu.make_async_copy(rows[b], acc.at[didx.at[j]], ss[b]).wait()
      if with_deg:
        pltpu.make_async_copy(ones, dacc.at[didx.at[j]], sd[b]).wait()

    for b in range(NBUF):
      gather(b, b)

    def group(g, carry):
      j0 = g * NBUF
      for b in range(NBUF):
        gather_wait(j0 + b, b)
        scatter(j0 + b, b)
      for b in range(NBUF):
        scatter_wait(j0 + b, b)
        gather(j0 + NBUF + b, b)
      return carry

    lax.fori_loop(0, NCHUNK // NBUF - 1, group, 0)

    j0 = NCHUNK - NBUF
    for b in range(NBUF):
      gather_wait(j0 + b, b)
      scatter(j0 + b, b)
    for b in range(NBUF):
      scatter_wait(j0 + b, b)
    plsc.subcore_barrier()

    fin = [pltpu.async_copy(acc.at[pl.ds(rbase, RPT)],
                            out_acc.at[pl.ds(c * NPAD + rbase, RPT)], sg[0])]
    if with_deg:
      fin.append(pltpu.async_copy(dacc.at[pl.ds(rbase, RPT)],
                                  out_deg.at[pl.ds(c * NPAD + rbase, RPT)],
                                  sg[1]))
    for cp in fin:
      cp.wait()

  return pl.kernel(
      body, out_type=out_type, mesh=mesh, scratch_types=scratch,
      compiler_params=pltpu.CompilerParams(use_tc_tiling_on_sc=False))


_agg_deg = _make_agg(with_deg=True)
_agg = _make_agg(with_deg=False)


ROWS = 5000


def _l1_body(x, a0, a1, d0, d1, ws, wn, b, o):
  deg = d0[...][:, 0:1] + d1[...][:, 0:1]
  inv = 1.0 / jnp.maximum(deg, 1.0)
  hn = (a0[...] + a1[...]) * inv
  s0 = jnp.dot(x[...], ws[...], preferred_element_type=f32)
  o[...] = jnp.maximum(
      s0 + jnp.dot(hn, wn[...], preferred_element_type=f32) + b[...], 0.0)


def _l2_body(h1, a0, a1, d0, d1, ws, wn, b, wfc, bfc, o):
  deg = d0[...][:, 0:1] + d1[...][:, 0:1]
  inv = 1.0 / jnp.maximum(deg, 1.0)
  hn = (a0[...] + a1[...]) * inv
  h2 = jnp.maximum(
      jnp.dot(h1[...], ws[...], preferred_element_type=f32)
      + jnp.dot(hn, wn[...], preferred_element_type=f32) + b[...], 0.0)
  o[...] = jnp.dot(h2, wfc[...], preferred_element_type=f32) + bfc[...]


def _row_spec(w):
  return pl.BlockSpec((ROWS, w), lambda i: (i, 0))


def _full_spec(shape):
  return pl.BlockSpec(shape, lambda i: tuple(0 for _ in shape))


def _tc_layer1(x, a0, a1, d0, d1, ws, wn, b):
  return pl.pallas_call(
      _l1_body,
      grid=(N // ROWS,),
      in_specs=[
          _row_spec(D), _row_spec(D), _row_spec(D),
          _row_spec(DEGW), _row_spec(DEGW),
          _full_spec((D, H)), _full_spec((D, H)), _full_spec((1, H)),
      ],
      out_specs=_row_spec(H),
      out_shape=jax.ShapeDtypeStruct((N, H), f32),
  )(x, a0, a1, d0, d1, ws, wn, b)


def _tc_layer2(h1, a0, a1, d0, d1, ws, wn, b, wfc, bfc):
  return pl.pallas_call(
      _l2_body,
      grid=(N // ROWS,),
      in_specs=[
          _row_spec(H), _row_spec(H), _row_spec(H),
          _row_spec(DEGW), _row_spec(DEGW),
          _full_spec((H, H)), _full_spec((H, H)), _full_spec((1, H)),
          _full_spec((H, C)), _full_spec((1, C)),
      ],
      out_specs=_row_spec(C),
      out_shape=jax.ShapeDtypeStruct((N, C), f32),
  )(h1, a0, a1, d0, d1, ws, wn, b, wfc, bfc)


def kernel(x, edge_index, W_self0, W_neigh0, b0, W_self1, W_neigh1, b1,
           W_fc, b_fc):
  src = edge_index[0]
  dst = edge_index[1]
  pad = EPAD - E
  iota = jnp.arange(pad, dtype=dst.dtype)
  src2d = jnp.concatenate([src, iota % N]).reshape(-1, CHUNK)
  dst2d = jnp.concatenate([dst, N + iota % (NPAD - N)]).reshape(-1, CHUNK)
  zacc = jnp.zeros((RPT, D), f32)
  zdacc = jnp.zeros((RPT, DEGW), f32)
  odeg = jnp.ones((CHUNK, DEGW), f32)

  aggx, deg = _agg_deg(x, src2d, dst2d, zacc, zdacc, odeg)
  h1 = _tc_layer1(x, aggx[:N], aggx[NPAD:NPAD + N], deg[:N],
                  deg[NPAD:NPAD + N], W_self0, W_neigh0, b0.reshape(1, H))
  aggh, = _agg(h1, src2d, dst2d, zacc)
  out = _tc_layer2(h1, aggh[:N], aggh[NPAD:NPAD + N], deg[:N],
                   deg[NPAD:NPAD + N], W_self1, W_neigh1, b1.reshape(1, H),
                   W_fc, b_fc.reshape(1, C))
  return out
